# baseline (speedup 1.0000x reference)
import functools

import jax
import jax.numpy as jnp
from jax import lax
from jax.experimental import pallas as pl
from jax.experimental.pallas import tpu as pltpu
from jax.experimental.pallas import tpu_sc as plsc

NTOK = 16384
EMB = 32
NC, NS = 2, 16
NW = NC * NS
BPW = NTOK // NW
CHUNK = 128
NCH = BPW // CHUNK

_mesh = plsc.VectorSubcoreMesh(
    core_axis_name="c", subcore_axis_name="s", num_cores=NC, num_subcores=NS
)


@functools.partial(
    pl.kernel,
    out_type=(
        jax.ShapeDtypeStruct((NTOK, EMB), jnp.float32),
        jax.ShapeDtypeStruct((NTOK, EMB), jnp.float32),
    ),
    mesh=_mesh,
    scratch_types=[
        pltpu.VMEM((BPW,), jnp.int32),
        pltpu.VMEM((BPW, EMB), jnp.float32),
        pltpu.SemaphoreType.DMA,
    ],
    compiler_params=pltpu.CompilerParams(use_tc_tiling_on_sc=False),
)
def _gather_kernel(tok_hbm, weight_hbm, out_hbm, out2_hbm, idx_v, rows_v, sem):
    wid = lax.axis_index("s") * NC + lax.axis_index("c")
    pltpu.sync_copy(tok_hbm.at[pl.ds(wid * BPW, BPW)], idx_v)
    copies = [
        pltpu.async_copy(
            weight_hbm.at[idx_v.at[pl.ds(j * CHUNK, CHUNK)]],
            rows_v.at[pl.ds(j * CHUNK, CHUNK)],
            sem,
        )
        for j in range(NCH)
    ]
    for c in copies:
        c.wait()
    pltpu.sync_copy(rows_v, out_hbm.at[pl.ds(wid * BPW, BPW)])
    pltpu.sync_copy(rows_v, out2_hbm.at[pl.ds(wid * BPW, BPW)])


def kernel(tokens, weight, bias):
    del bias
    return _gather_kernel(tokens, weight)


# speedup vs baseline: 1.0206x; 1.0206x over previous
import functools

import jax
import jax.numpy as jnp
from jax import lax
from jax.experimental import pallas as pl
from jax.experimental.pallas import tpu as pltpu
from jax.experimental.pallas import tpu_sc as plsc

NTOK = 16384
EMB = 32
NC, NS = 2, 16
NW = NC * NS
BPW = NTOK // NW
CHUNK = 128
NCH = BPW // CHUNK

_mesh = plsc.VectorSubcoreMesh(
    core_axis_name="c", subcore_axis_name="s", num_cores=NC, num_subcores=NS
)


@functools.partial(
    pl.kernel,
    out_type=jax.ShapeDtypeStruct((NTOK, EMB), jnp.float32),
    mesh=_mesh,
    scratch_types=[
        pltpu.VMEM((NCH, CHUNK), jnp.int32),
        pltpu.VMEM((BPW, EMB), jnp.float32),
        pltpu.SemaphoreType.DMA,
    ],
    compiler_params=pltpu.CompilerParams(use_tc_tiling_on_sc=False),
)
def _gather_kernel(tok_hbm, weight_hbm, out_hbm, idx_v, rows_v, sem):
    wid = lax.axis_index("s") * NC + lax.axis_index("c")
    pltpu.sync_copy(tok_hbm.at[wid], idx_v)
    copies = [
        pltpu.async_copy(
            weight_hbm.at[idx_v.at[j]], rows_v.at[pl.ds(j * CHUNK, CHUNK)], sem
        )
        for j in range(NCH)
    ]
    for c in copies:
        c.wait()
    pltpu.sync_copy(rows_v, out_hbm.at[pl.ds(wid * BPW, BPW)])


def kernel(tokens, weight, bias):
    del bias
    tok3 = tokens.reshape(NW, NCH, CHUNK)
    out = _gather_kernel(tok3, weight)
    return (out, out)


# speedup vs baseline: 3.2087x; 3.1440x over previous
import functools

import jax
import jax.numpy as jnp
from jax import lax
from jax.experimental import pallas as pl
from jax.experimental.pallas import tpu as pltpu
from jax.experimental.pallas import tpu_sc as plsc

NTOK = 16384
EMB = 32
VOCAB = 1000000
NC, NS = 2, 16
NW = NC * NS
BPW = NTOK // NW
BATCH = 16
NB = BPW // BATCH
ORPW = BPW * EMB // 128

_mesh = plsc.VectorSubcoreMesh(
    core_axis_name="c", subcore_axis_name="s", num_cores=NC, num_subcores=NS
)

_OUT = jax.ShapeDtypeStruct((NTOK * EMB // 128, 128), jnp.float32)


@functools.partial(
    pl.kernel,
    out_type=(_OUT, _OUT),
    mesh=_mesh,
    scratch_types=[
        pltpu.VMEM((BPW,), jnp.int32),
        pltpu.VMEM((BPW,), jnp.int32),
        pltpu.VMEM((BATCH, EMB, 128), jnp.float32),
        pltpu.VMEM((ORPW, 128), jnp.float32),
        pltpu.SemaphoreType.DMA,
    ],
    compiler_params=pltpu.CompilerParams(
        use_tc_tiling_on_sc=True, needs_layout_passes=False
    ),
)
def _gather_kernel(tok_hbm, wT_hbm, out_hbm, out2_hbm,
                   tok_v, col_v, tile_v, rows_v, sem):
    wid = lax.axis_index("s") * NC + lax.axis_index("c")
    base = wid * BPW
    pltpu.sync_copy(tok_hbm.at[pl.ds(base, BPW)], tok_v)

    w3 = wT_hbm.reshape((4, 8, VOCAB))
    lane = lax.iota(jnp.int32, 16)

    def colidx(c, carry):
        t16 = tok_v[pl.ds(c * 16, 16)]
        col_v[pl.ds(c * 16, 16)] = (t16 >> 7) << 7
        return carry

    lax.fori_loop(0, BPW // 16, colidx, 0)

    def batch_body(k, carry):
        j16 = col_v[pl.ds(k * BATCH, 16)]
        descs = []
        for s in range(BATCH):
            j0 = pl.multiple_of(j16[s], 128)
            for g in range(4):
                descs.append(
                    pltpu.async_copy(
                        w3.at[g, :, pl.ds(j0, 128)],
                        tile_v.at[s, pl.ds(g * 8, 8)],
                        sem,
                    )
                )
        for d in descs:
            d.wait()

        t16 = tok_v[pl.ds(k * BATCH, 16)]
        c16 = t16 & 127
        m16 = k * BATCH + lane
        for e in range(EMB):
            vals = plsc.load_gather(
                tile_v, [lane, jnp.full((16,), e, jnp.int32), c16]
            )
            p16 = m16 * EMB + e
            plsc.store_scatter(rows_v, [p16 >> 7, p16 & 127], vals)
        return carry

    lax.fori_loop(0, NB, batch_body, 0)

    pltpu.sync_copy(rows_v, out_hbm.at[pl.ds(wid * ORPW, ORPW)])
    pltpu.sync_copy(rows_v, out2_hbm.at[pl.ds(wid * ORPW, ORPW)])


def kernel(tokens, weight, bias):
    del bias
    o1, o2 = _gather_kernel(tokens, weight.T)
    return (o1.reshape(NTOK, EMB), o2.reshape(NTOK, EMB))
